# Initial kernel scaffold; baseline (speedup 1.0000x reference)
import jax
import jax.numpy as jnp
from jax.experimental import pallas as pl


def kernel(y, sigma, x, x_influences):
    raise NotImplementedError("write your pallas kernel here")


# speedup vs baseline: 1.2558x; 1.2558x over previous
import functools

import jax
import jax.numpy as jnp
from jax import lax
from jax.experimental import pallas as pl
from jax.experimental.pallas import tpu as pltpu
from jax.experimental.pallas import tpu_sc as plsc

B = 131072
C = 128
L = 16
NC = 2
NS = 16
NW = NC * NS
CHUNK = 256
VROWS = CHUNK * C // L
GROUPS = CHUNK // L
CHUNKS_PER_W = B // NW // CHUNK


def _sc_body(y_h, s_h, x_h, infl_h, o_h, ybuf, sbuf, obuf, xbuf, inflbuf):
    wid = lax.axis_index("s") * NC + lax.axis_index("c")
    pltpu.sync_copy(infl_h, inflbuf)
    lanes = lax.iota(jnp.int32, L)

    def chunk_body(k, _):
        cidx = wid * CHUNKS_PER_W + k
        rbase = cidx * VROWS
        xbase = cidx * GROUPS
        pltpu.sync_copy(y_h.at[pl.ds(rbase, VROWS)], ybuf)
        pltpu.sync_copy(s_h.at[pl.ds(rbase, VROWS)], sbuf)
        pltpu.sync_copy(x_h.at[pl.ds(xbase, GROUPS)], xbuf)

        def dense_body(i, carry):
            yv = ybuf[i, :]
            sv = sbuf[i, :]
            obuf[i, :] = (0.0 - yv) / (sv * sv)
            return carry

        lax.fori_loop(0, VROWS, dense_body, 0)

        def group_body(g, carry):
            xv = xbuf[g, :]
            rows8 = (g * L + lanes) * (C // L)

            def col_body(c, st):
                best, bidx = st
                i0 = rows8 + (c >> 4)
                i1 = jnp.full((L,), c & 15, jnp.int32)
                yc = plsc.load_gather(ybuf, [i0, i1])
                ic = plsc.load_gather(
                    inflbuf,
                    [jnp.full((L,), c >> 4, jnp.int32), i1],
                )
                s = xv * ic
                m = jnp.where(s < 0.0, yc - 1e32, yc)
                upd = m > best
                best = jnp.where(upd, m, best)
                bidx = jnp.where(upd, jnp.full((L,), c, jnp.int32), bidx)
                return best, bidx

            init = (jnp.full((L,), -jnp.inf, jnp.float32),
                    jnp.zeros((L,), jnp.int32))
            _, bidx = lax.fori_loop(0, C, col_body, init)

            j0 = rows8 + (bidx >> 4)
            j1 = bidx & 15
            yat = plsc.load_gather(ybuf, [j0, j1])
            sat = plsc.load_gather(sbuf, [j0, j1])
            fv = (1.0 - yat) / (sat * sat)
            plsc.store_scatter(obuf, [j0, j1], fv)
            return carry

        lax.fori_loop(0, GROUPS, group_body, 0)

        pltpu.sync_copy(obuf, o_h.at[pl.ds(rbase, VROWS)])
        return _

    lax.fori_loop(0, CHUNKS_PER_W, chunk_body, 0)


@functools.partial(
    pl.kernel,
    out_type=jax.ShapeDtypeStruct((B * C // L, L), jnp.float32),
    mesh=plsc.VectorSubcoreMesh(core_axis_name="c", subcore_axis_name="s"),
    compiler_params=pltpu.CompilerParams(
        needs_layout_passes=False, use_tc_tiling_on_sc=False
    ),
    scratch_types=[
        pltpu.VMEM((VROWS, L), jnp.float32),
        pltpu.VMEM((VROWS, L), jnp.float32),
        pltpu.VMEM((VROWS, L), jnp.float32),
        pltpu.VMEM((GROUPS, L), jnp.float32),
        pltpu.VMEM((C // L, L), jnp.float32),
    ],
)
def _sc_kernel(y_h, s_h, x_h, infl_h, o_h, ybuf, sbuf, obuf, xbuf, inflbuf):
    _sc_body(y_h, s_h, x_h, infl_h, o_h, ybuf, sbuf, obuf, xbuf, inflbuf)


@jax.jit
def kernel(y, sigma, x, x_influences):
    out2 = _sc_kernel(
        y.reshape(-1, L),
        sigma.reshape(-1, L),
        x.reshape(-1, L),
        x_influences.reshape(-1, L),
    )
    return out2.reshape(B, C)


# speedup vs baseline: 1.5917x; 1.2675x over previous
import functools

import jax
import jax.numpy as jnp
from jax import lax
from jax.experimental import pallas as pl
from jax.experimental.pallas import tpu as pltpu
from jax.experimental.pallas import tpu_sc as plsc

B = 131072
C = 128
L = 16
NC = 2
NS = 16
NW = NC * NS
CHUNK = 256
VROWS = CHUNK * C // L
GROUPS = CHUNK // L
CHUNKS_PER_W = B // NW // CHUNK


def _sc_body(y_h, s_h, x_h, infl_h, o_h, ybuf, sbuf, obuf, xbuf, inflbuf):
    wid = lax.axis_index("s") * NC + lax.axis_index("c")
    pltpu.sync_copy(infl_h, inflbuf)
    lanes = lax.iota(jnp.int32, L)

    def chunk_body(k, _):
        cidx = wid * CHUNKS_PER_W + k
        rbase = cidx * VROWS
        xbase = cidx * GROUPS
        pltpu.sync_copy(y_h.at[pl.ds(rbase, VROWS)], ybuf)
        pltpu.sync_copy(s_h.at[pl.ds(rbase, VROWS)], sbuf)
        pltpu.sync_copy(x_h.at[pl.ds(xbase, GROUPS)], xbuf)

        @plsc.parallel_loop(0, VROWS, unroll=8)
        def dense_body(i):
            yv = ybuf[i, :]
            sv = sbuf[i, :]
            obuf[i, :] = (0.0 - yv) / (sv * sv)

        @plsc.parallel_loop(0, GROUPS)
        def group_body(g):
            xv = xbuf[g, :]
            rows8 = (g * L + lanes) * (C // L)

            init = (jnp.full((L,), -jnp.inf, jnp.float32),
                    jnp.zeros((L,), jnp.int32))

            @plsc.parallel_loop(0, C, unroll=8, carry=init)
            def col_body(c, st):
                best, bidx = st
                i0 = rows8 + (c >> 4)
                i1 = jnp.full((L,), c & 15, jnp.int32)
                yc = plsc.load_gather(ybuf, [i0, i1])
                ic = plsc.load_gather(
                    inflbuf,
                    [jnp.full((L,), c >> 4, jnp.int32), i1],
                )
                s = xv * ic
                m = jnp.where(s < 0.0, yc - 1e32, yc)
                upd = m > best
                best = jnp.where(upd, m, best)
                bidx = jnp.where(upd, jnp.full((L,), c, jnp.int32), bidx)
                return best, bidx

            _, bidx = col_body

            j0 = rows8 + (bidx >> 4)
            j1 = bidx & 15
            yat = plsc.load_gather(ybuf, [j0, j1])
            sat = plsc.load_gather(sbuf, [j0, j1])
            fv = (1.0 - yat) / (sat * sat)
            plsc.store_scatter(obuf, [j0, j1], fv)

        pltpu.sync_copy(obuf, o_h.at[pl.ds(rbase, VROWS)])
        return _

    lax.fori_loop(0, CHUNKS_PER_W, chunk_body, 0)


@functools.partial(
    pl.kernel,
    out_type=jax.ShapeDtypeStruct((B * C // L, L), jnp.float32),
    mesh=plsc.VectorSubcoreMesh(core_axis_name="c", subcore_axis_name="s"),
    compiler_params=pltpu.CompilerParams(
        needs_layout_passes=False, use_tc_tiling_on_sc=False
    ),
    scratch_types=[
        pltpu.VMEM((VROWS, L), jnp.float32),
        pltpu.VMEM((VROWS, L), jnp.float32),
        pltpu.VMEM((VROWS, L), jnp.float32),
        pltpu.VMEM((GROUPS, L), jnp.float32),
        pltpu.VMEM((C // L, L), jnp.float32),
    ],
)
def _sc_kernel(y_h, s_h, x_h, infl_h, o_h, ybuf, sbuf, obuf, xbuf, inflbuf):
    _sc_body(y_h, s_h, x_h, infl_h, o_h, ybuf, sbuf, obuf, xbuf, inflbuf)


@jax.jit
def kernel(y, sigma, x, x_influences):
    out2 = _sc_kernel(
        y.reshape(-1, L),
        sigma.reshape(-1, L),
        x.reshape(-1, L),
        x_influences.reshape(-1, L),
    )
    return out2.reshape(B, C)
